# baseline (speedup 1.0000x reference)
import jax
import jax.numpy as jnp
from jax import lax
from jax.experimental import pallas as pl
from jax.experimental.pallas import tpu as pltpu
from jax.experimental.pallas import tpu_sc as plsc

N_NODES = 10000
N_EDGES = 320000
DIM = 128

NC = 2
NS = 16
NW = NC * NS

EC = 64
CHUNKS_PER_W = 157
N_CHUNKS = NW * CHUNKS_PER_W
E_PAD = N_CHUNKS * EC
N_TAB = 10112
ROWS_PER_SUB = N_TAB // NS


def _proj_body(x_ref, we_ref, be_ref, ps_ref, pd_ref):
    xv = x_ref[...]
    ps_ref[...] = jnp.dot(xv, we_ref[0:DIM, :], preferred_element_type=jnp.float32)
    pd_ref[...] = (
        jnp.dot(xv, we_ref[DIM : 2 * DIM, :], preferred_element_type=jnp.float32)
        + be_ref[...]
    )


def _project(x_pad, W_edge, b_edge):
    return pl.pallas_call(
        _proj_body,
        out_shape=[
            jax.ShapeDtypeStruct((N_TAB, DIM), jnp.float32),
            jax.ShapeDtypeStruct((N_TAB, DIM), jnp.float32),
        ],
    )(x_pad, W_edge, b_edge.reshape(1, DIM))


def _sc_body(ps_hbm, pd_hbm, ei_hbm, out_hbm,
             idx, rows_a, rows_b, rows_m, agg,
             gsem0, gsem1, ssem0, ssem1, isem):
    cid = lax.axis_index("c")
    sid = lax.axis_index("s")
    wid = sid * NC + cid
    gsems = (gsem0, gsem1)
    ssems = (ssem0, ssem1)

    @pl.loop(0, EC)
    def _zero_rows(j):
        for i in range(DIM // 16):
            rows_m[0, j, pl.ds(i * 16, 16)] = jnp.zeros((16,), jnp.float32)

    row0 = sid * ROWS_PER_SUB
    for r in range(ROWS_PER_SUB // EC):
        pltpu.sync_copy(rows_m.at[0], agg.at[pl.ds(row0 + r * EC, EC)])
    rem = ROWS_PER_SUB % EC
    if rem:
        pltpu.sync_copy(rows_m.at[0, pl.ds(0, rem)],
                        agg.at[pl.ds(row0 + ROWS_PER_SUB - rem, rem)])
    plsc.subcore_barrier()

    def _issue_idx(t, q):
        pltpu.async_copy(ei_hbm.at[wid * CHUNKS_PER_W + t], idx.at[q], isem)

    def _wait_idx(q):
        pltpu.make_async_copy(ei_hbm.at[0], idx.at[q], isem).wait()

    def _issue_gathers(p, q):
        pltpu.async_copy(ps_hbm.at[idx.at[q, 0]], rows_a.at[p], gsems[p])
        pltpu.async_copy(pd_hbm.at[idx.at[q, 1]], rows_b.at[p], gsems[p])

    def _wait_gathers(p, q):
        pltpu.make_async_copy(ps_hbm.at[idx.at[q, 0]], rows_a.at[p], gsems[p]).wait()
        pltpu.make_async_copy(pd_hbm.at[idx.at[q, 1]], rows_b.at[p], gsems[p]).wait()

    def _issue_scatter(p, q):
        pltpu.async_copy(rows_m.at[p], agg.at[idx.at[q, 1]], ssems[p], add=True)

    def _wait_scatter(p, q):
        pltpu.make_async_copy(rows_m.at[p], agg.at[idx.at[q, 1]], ssems[p]).wait()

    def _compute(p):
        @plsc.parallel_loop(0, EC, unroll=8)
        def _relu_add(j):
            for i in range(DIM // 16):
                a = rows_a[p, j, pl.ds(i * 16, 16)]
                b = rows_b[p, j, pl.ds(i * 16, 16)]
                rows_m[p, j, pl.ds(i * 16, 16)] = jnp.maximum(a + b, 0.0)

    def _process(t, p, first=False, last=False):
        q = t % 4
        qn = (t + 2) % 4
        _wait_gathers(p, q)
        if not first:
            _wait_scatter(p, qn)
        if not last:
            _issue_idx(t + 2, qn)
        _compute(p)
        _issue_scatter(p, q)
        if not last:
            _wait_idx(qn)
            _issue_gathers(p, qn)

    _issue_idx(0, 0)
    _issue_idx(1, 1)
    _wait_idx(0)
    _issue_gathers(0, 0)
    _wait_idx(1)
    _issue_gathers(1, 1)
    _process(0, 0, first=True)
    _process(1, 1, first=True)

    @pl.loop(2, CHUNKS_PER_W - 3, step=2)
    def _pair(t):
        _process(t, 0)
        _process(t + 1, 1)

    _process(CHUNKS_PER_W - 3, 0)
    _process(CHUNKS_PER_W - 2, 1, last=True)
    _process(CHUNKS_PER_W - 1, 0, last=True)
    _wait_scatter((CHUNKS_PER_W - 2) % 2, (CHUNKS_PER_W - 2) % 4)
    _wait_scatter((CHUNKS_PER_W - 1) % 2, (CHUNKS_PER_W - 1) % 4)
    plsc.subcore_barrier()

    for r in range(ROWS_PER_SUB // EC):
        off = row0 + r * EC
        pltpu.sync_copy(agg.at[pl.ds(off, EC)],
                        out_hbm.at[pl.ds(cid * N_TAB + off, EC)])
    if rem:
        off = row0 + ROWS_PER_SUB - rem
        pltpu.sync_copy(agg.at[pl.ds(off, rem)],
                        out_hbm.at[pl.ds(cid * N_TAB + off, rem)])


def _sc_edge_stage(ps, pd, ei_blocks):
    mesh = plsc.VectorSubcoreMesh(
        core_axis_name="c", subcore_axis_name="s", num_cores=NC, num_subcores=NS
    )
    return pl.kernel(
        _sc_body,
        out_type=jax.ShapeDtypeStruct((NC * N_TAB, DIM), jnp.float32),
        mesh=mesh,
        scratch_types=[
            pltpu.VMEM((4, 2, EC), jnp.int32),
            pltpu.VMEM((2, EC, DIM), jnp.float32),
            pltpu.VMEM((2, EC, DIM), jnp.float32),
            pltpu.VMEM((2, EC, DIM), jnp.float32),
            pltpu.VMEM_SHARED((N_TAB, DIM), jnp.float32),
            pltpu.SemaphoreType.DMA,
            pltpu.SemaphoreType.DMA,
            pltpu.SemaphoreType.DMA,
            pltpu.SemaphoreType.DMA,
            pltpu.SemaphoreType.DMA,
        ],
    )(ps, pd, ei_blocks)


def _final_body(x_ref, agg_ref, wn_ref, bn_ref, o_ref):
    a = agg_ref[0:N_NODES, :] + agg_ref[N_TAB : N_TAB + N_NODES, :]
    o_ref[...] = (
        jnp.dot(x_ref[...], wn_ref[0:DIM, :], preferred_element_type=jnp.float32)
        + jnp.dot(a, wn_ref[DIM : 2 * DIM, :], preferred_element_type=jnp.float32)
        + bn_ref[...]
    )


def _final(x, agg2, W_node, b_node):
    return pl.pallas_call(
        _final_body,
        out_shape=jax.ShapeDtypeStruct((N_NODES, DIM), jnp.float32),
    )(x, agg2, W_node, b_node.reshape(1, DIM))


def kernel(x, edge_index, W_edge, b_edge, W_node, b_node):
    src = edge_index[0].astype(jnp.int32)
    dst = edge_index[1].astype(jnp.int32)
    n_dummy = E_PAD - N_EDGES
    ar = jnp.arange(n_dummy, dtype=jnp.int32)
    src_p = jnp.concatenate([src, ar % N_NODES])
    dst_p = jnp.concatenate([dst, N_NODES + ar % (N_TAB - N_NODES)])
    ei_blocks = jnp.stack(
        [src_p.reshape(N_CHUNKS, EC), dst_p.reshape(N_CHUNKS, EC)], axis=1
    )
    x_pad = jnp.pad(x, ((0, N_TAB - N_NODES), (0, 0)))

    ps, pd = _project(x_pad, W_edge, b_edge)
    agg2 = _sc_edge_stage(ps, pd, ei_blocks)
    return _final(x, agg2, W_node, b_node)


# speedup vs baseline: 1.0440x; 1.0440x over previous
import jax
import jax.numpy as jnp
from jax import lax
from jax.experimental import pallas as pl
from jax.experimental.pallas import tpu as pltpu
from jax.experimental.pallas import tpu_sc as plsc

N_NODES = 10000
N_EDGES = 320000
DIM = 128

NC = 2
NS = 16
NW = NC * NS

EC = 64
CHUNKS_PER_W = 157
N_CHUNKS = NW * CHUNKS_PER_W
E_PAD = N_CHUNKS * EC
N_TAB = 10112
ROWS_PER_SUB = N_TAB // NS


def _proj_body(x_ref, we_ref, be_ref, ps_ref, pd_ref):
    xv = x_ref[...]
    ps_ref[...] = jnp.dot(xv, we_ref[0:DIM, :], preferred_element_type=jnp.float32)
    pd_ref[...] = (
        jnp.dot(xv, we_ref[DIM : 2 * DIM, :], preferred_element_type=jnp.float32)
        + be_ref[...]
    )


def _project(x_pad, W_edge, b_edge):
    return pl.pallas_call(
        _proj_body,
        out_shape=[
            jax.ShapeDtypeStruct((N_TAB, DIM), jnp.float32),
            jax.ShapeDtypeStruct((N_TAB, DIM), jnp.float32),
        ],
    )(x_pad, W_edge, b_edge.reshape(1, DIM))


def _sc_body(ps_hbm, pd_hbm, ei_hbm, out_hbm,
             idx, rows_a, rows_b, rows_m, agg,
             gsem0, gsem1, ssem0, ssem1, isem):
    cid = lax.axis_index("c")
    sid = lax.axis_index("s")
    wid = sid * NC + cid
    gsems = (gsem0, gsem1)
    ssems = (ssem0, ssem1)

    @pl.loop(0, EC)
    def _zero_rows(j):
        for i in range(DIM // 16):
            rows_m[0, j, pl.ds(i * 16, 16)] = jnp.zeros((16,), jnp.float32)

    row0 = sid * ROWS_PER_SUB
    for r in range(ROWS_PER_SUB // EC):
        pltpu.sync_copy(rows_m.at[0], agg.at[pl.ds(row0 + r * EC, EC)])
    rem = ROWS_PER_SUB % EC
    if rem:
        pltpu.sync_copy(rows_m.at[0, pl.ds(0, rem)],
                        agg.at[pl.ds(row0 + ROWS_PER_SUB - rem, rem)])
    plsc.subcore_barrier()

    def _issue_idx(t, q):
        pltpu.async_copy(ei_hbm.at[wid * CHUNKS_PER_W + t], idx.at[q], isem)

    def _wait_idx(q):
        pltpu.make_async_copy(ei_hbm.at[0], idx.at[q], isem).wait()

    def _issue_gathers(p, q):
        pltpu.async_copy(ps_hbm.at[idx.at[q, 0]], rows_a.at[p], gsems[p])
        pltpu.async_copy(pd_hbm.at[idx.at[q, 1]], rows_b.at[p], gsems[p])

    def _wait_gathers(p, q):
        pltpu.make_async_copy(ps_hbm.at[idx.at[q, 0]], rows_a.at[p], gsems[p]).wait()
        pltpu.make_async_copy(pd_hbm.at[idx.at[q, 1]], rows_b.at[p], gsems[p]).wait()

    def _issue_scatter(p, q):
        pltpu.async_copy(rows_m.at[p], agg.at[idx.at[q, 1]], ssems[p], add=True)

    def _wait_scatter(p, q):
        pltpu.make_async_copy(rows_m.at[p], agg.at[idx.at[q, 1]], ssems[p]).wait()

    def _compute(p):
        @plsc.parallel_loop(0, EC, unroll=4)
        def _relu_add(j):
            for i in range(DIM // 16):
                a = rows_a[p, j, pl.ds(i * 16, 16)]
                b = rows_b[p, j, pl.ds(i * 16, 16)]
                rows_m[p, j, pl.ds(i * 16, 16)] = jnp.maximum(a + b, 0.0)

    def _process(t, p, first=False, last=False):
        q = t % 4
        qn = (t + 2) % 4
        _wait_gathers(p, q)
        if not first:
            _wait_scatter(p, qn)
        if not last:
            _issue_idx(t + 2, qn)
        _compute(p)
        _issue_scatter(p, q)
        if not last:
            _wait_idx(qn)
            _issue_gathers(p, qn)

    _issue_idx(0, 0)
    _issue_idx(1, 1)
    _wait_idx(0)
    _issue_gathers(0, 0)
    _wait_idx(1)
    _issue_gathers(1, 1)
    _process(0, 0, first=True)
    _process(1, 1, first=True)

    @pl.loop(2, CHUNKS_PER_W - 3, step=2)
    def _pair(t):
        _process(t, 0)
        _process(t + 1, 1)

    _process(CHUNKS_PER_W - 3, 0)
    _process(CHUNKS_PER_W - 2, 1, last=True)
    _process(CHUNKS_PER_W - 1, 0, last=True)
    _wait_scatter((CHUNKS_PER_W - 2) % 2, (CHUNKS_PER_W - 2) % 4)
    _wait_scatter((CHUNKS_PER_W - 1) % 2, (CHUNKS_PER_W - 1) % 4)
    plsc.subcore_barrier()

    for r in range(ROWS_PER_SUB // EC):
        off = row0 + r * EC
        pltpu.sync_copy(agg.at[pl.ds(off, EC)],
                        out_hbm.at[pl.ds(cid * N_TAB + off, EC)])
    if rem:
        off = row0 + ROWS_PER_SUB - rem
        pltpu.sync_copy(agg.at[pl.ds(off, rem)],
                        out_hbm.at[pl.ds(cid * N_TAB + off, rem)])


def _sc_edge_stage(ps, pd, ei_blocks):
    mesh = plsc.VectorSubcoreMesh(
        core_axis_name="c", subcore_axis_name="s", num_cores=NC, num_subcores=NS
    )
    return pl.kernel(
        _sc_body,
        out_type=jax.ShapeDtypeStruct((NC * N_TAB, DIM), jnp.float32),
        mesh=mesh,
        scratch_types=[
            pltpu.VMEM((4, 2, EC), jnp.int32),
            pltpu.VMEM((2, EC, DIM), jnp.float32),
            pltpu.VMEM((2, EC, DIM), jnp.float32),
            pltpu.VMEM((2, EC, DIM), jnp.float32),
            pltpu.VMEM_SHARED((N_TAB, DIM), jnp.float32),
            pltpu.SemaphoreType.DMA,
            pltpu.SemaphoreType.DMA,
            pltpu.SemaphoreType.DMA,
            pltpu.SemaphoreType.DMA,
            pltpu.SemaphoreType.DMA,
        ],
    )(ps, pd, ei_blocks)


def _final_body(x_ref, agg_ref, wn_ref, bn_ref, o_ref):
    a = agg_ref[0:N_NODES, :] + agg_ref[N_TAB : N_TAB + N_NODES, :]
    o_ref[...] = (
        jnp.dot(x_ref[...], wn_ref[0:DIM, :], preferred_element_type=jnp.float32)
        + jnp.dot(a, wn_ref[DIM : 2 * DIM, :], preferred_element_type=jnp.float32)
        + bn_ref[...]
    )


def _final(x, agg2, W_node, b_node):
    return pl.pallas_call(
        _final_body,
        out_shape=jax.ShapeDtypeStruct((N_NODES, DIM), jnp.float32),
    )(x, agg2, W_node, b_node.reshape(1, DIM))


def kernel(x, edge_index, W_edge, b_edge, W_node, b_node):
    src = edge_index[0].astype(jnp.int32)
    dst = edge_index[1].astype(jnp.int32)
    n_dummy = E_PAD - N_EDGES
    ar = jnp.arange(n_dummy, dtype=jnp.int32)
    src_p = jnp.concatenate([src, ar % N_NODES])
    dst_p = jnp.concatenate([dst, N_NODES + ar % (N_TAB - N_NODES)])
    ei_blocks = jnp.stack(
        [src_p.reshape(N_CHUNKS, EC), dst_p.reshape(N_CHUNKS, EC)], axis=1
    )
    x_pad = jnp.pad(x, ((0, N_TAB - N_NODES), (0, 0)))

    ps, pd = _project(x_pad, W_edge, b_edge)
    agg2 = _sc_edge_stage(ps, pd, ei_blocks)
    return _final(x, agg2, W_node, b_node)
